# baseline (speedup 1.0000x reference)
import functools

import jax
import jax.numpy as jnp
import numpy as np
from jax import lax
from jax.experimental import pallas as pl
from jax.experimental.pallas import tpu as pltpu
from jax.experimental.pallas import tpu_sc as plsc

N_NODES = 10000
N_EDGES = 160000
IN_DIM = 32
OUT_DIM = 32
NEF = 16
HID = 16
ALPHA = float(1.0 / np.sqrt(IN_DIM * 1))
EPS = float(jnp.finfo(jnp.float32).eps)

NC = 2
NS = 16
NW = NC * NS
EW = N_EDGES // NW
GC = 1000
NCHUNK = EW // GC
STRIPE = N_NODES // NS

@functools.lru_cache(maxsize=None)
def _get_mesh():
    return plsc.VectorSubcoreMesh(core_axis_name="c", subcore_axis_name="s",
                                  num_cores=NC, num_subcores=NS)


def _sc_gather_body(node_hbm, dst_hbm, out_hbm, idx_v, rows_v, sem):
    wid = lax.axis_index("s") * NC + lax.axis_index("c")
    for i in range(NCHUNK):
        base = wid * EW + i * GC
        pltpu.sync_copy(dst_hbm.at[pl.ds(base, GC)], idx_v)
        pltpu.async_copy(node_hbm.at[idx_v], rows_v, sem).wait()
        pltpu.sync_copy(rows_v, out_hbm.at[pl.ds(base, GC)])


@functools.lru_cache(maxsize=None)
def _sc_gather():
    return pl.kernel(
        _sc_gather_body,
        out_type=jax.ShapeDtypeStruct((N_EDGES, IN_DIM), jnp.float32),
        mesh=_get_mesh(),
        scratch_types=[
            pltpu.VMEM((GC,), jnp.int32),
            pltpu.VMEM((GC, IN_DIM), jnp.float32),
            pltpu.SemaphoreType.DMA,
        ],
        compiler_params=pltpu.CompilerParams(use_tc_tiling_on_sc=False,
                                             needs_layout_passes=False),
    )


def _sc_scatter_body(t0, t1, t2, t3, src_hbm, z8_hbm, z16_hbm, ones_hbm,
                     psum_hbm, pcnt_hbm,
                     idx_v, v0, v1, v2, v3, ones_v,
                     a0, a1, a2, a3, shared_cnt):
    cid = lax.axis_index("c")
    sid = lax.axis_index("s")
    slabs = (t0, t1, t2, t3)
    vals = (v0, v1, v2, v3)
    accs = (a0, a1, a2, a3)
    row0 = sid * STRIPE
    for s in range(4):
        pltpu.sync_copy(z8_hbm.at[pl.ds(row0, STRIPE)],
                        accs[s].at[pl.ds(row0, STRIPE)])
    pltpu.sync_copy(z16_hbm.at[pl.ds(row0, STRIPE)],
                    shared_cnt.at[pl.ds(row0, STRIPE)])
    pltpu.sync_copy(ones_hbm, ones_v)
    plsc.subcore_barrier()
    wid = sid * NC + cid
    for i in range(NCHUNK):
        base = wid * EW + i * GC
        pltpu.sync_copy(src_hbm.at[pl.ds(base, GC)], idx_v)
        for s in range(4):
            pltpu.sync_copy(slabs[s].at[pl.ds(base, GC)], vals[s])
        for s in range(4):
            pltpu.sync_copy(vals[s], accs[s].at[idx_v], add=True)
        pltpu.sync_copy(ones_v, shared_cnt.at[idx_v], add=True)
    plsc.subcore_barrier()
    for s in range(4):
        pltpu.sync_copy(accs[s].at[pl.ds(row0, STRIPE)],
                        psum_hbm.at[cid, pl.ds(row0, STRIPE),
                                    pl.ds(8 * s, 8)])
    pltpu.sync_copy(shared_cnt.at[pl.ds(row0, STRIPE)],
                    pcnt_hbm.at[cid, pl.ds(row0, STRIPE)])


@functools.lru_cache(maxsize=None)
def _sc_scatter():
    return pl.kernel(
        _sc_scatter_body,
        out_type=(
            jax.ShapeDtypeStruct((NC, N_NODES, OUT_DIM), jnp.float32),
            jax.ShapeDtypeStruct((NC, N_NODES, HID), jnp.float32),
        ),
        mesh=_get_mesh(),
        scratch_types=[
            pltpu.VMEM((GC,), jnp.int32),
            pltpu.VMEM((GC, 8), jnp.float32),
            pltpu.VMEM((GC, 8), jnp.float32),
            pltpu.VMEM((GC, 8), jnp.float32),
            pltpu.VMEM((GC, 8), jnp.float32),
            pltpu.VMEM((GC, HID), jnp.float32),
            pltpu.VMEM_SHARED((N_NODES, 8), jnp.float32),
            pltpu.VMEM_SHARED((N_NODES, 8), jnp.float32),
            pltpu.VMEM_SHARED((N_NODES, 8), jnp.float32),
            pltpu.VMEM_SHARED((N_NODES, 8), jnp.float32),
            pltpu.VMEM_SHARED((N_NODES, HID), jnp.float32),
        ],
        compiler_params=pltpu.CompilerParams(use_tc_tiling_on_sc=False,
                                             needs_layout_passes=False),
    )


EB = 3200


def _dense_body(eaT_ref, x0_ref, x1_ref, x2_ref, x3_ref, shT_ref, w1_ref,
                b1c_ref, w2mT_ref, b2rT_ref,
                o0_ref, o1_ref, o2_ref, o3_ref):
    hT = jnp.maximum(
        jnp.dot(w1_ref[...], eaT_ref[...],
                preferred_element_type=jnp.float32) + b1c_ref[...], 0.0)
    xdT = jnp.concatenate(
        [x0_ref[...], x1_ref[...], x2_ref[...], x3_ref[...]], axis=0)
    xdsT = shT_ref[...] * xdT
    gT = jnp.dot(w2mT_ref[...], xdsT, preferred_element_type=jnp.float32)
    acc = jnp.dot(b2rT_ref[...], xdsT, preferred_element_type=jnp.float32)
    for j in range(HID):
        acc = acc + gT[j * OUT_DIM:(j + 1) * OUT_DIM, :] * hT[j:j + 1, :]
    o0_ref[...] = acc[0:8, :]
    o1_ref[...] = acc[8:16, :]
    o2_ref[...] = acc[16:24, :]
    o3_ref[...] = acc[24:32, :]


def _dense(eaT, xs, shT, w1, b1c, w2mT, b2rT):
    slab_spec = pl.BlockSpec((8, EB), lambda i: (0, i))
    return pl.pallas_call(
        _dense_body,
        grid=(N_EDGES // EB,),
        in_specs=[
            pl.BlockSpec((NEF, EB), lambda i: (0, i)),
            slab_spec, slab_spec, slab_spec, slab_spec,
            pl.BlockSpec((1, EB), lambda i: (0, i)),
            pl.BlockSpec((NEF, NEF), lambda i: (0, 0)),
            pl.BlockSpec((HID, 1), lambda i: (0, 0)),
            pl.BlockSpec((HID * OUT_DIM, IN_DIM), lambda i: (0, 0)),
            pl.BlockSpec((OUT_DIM, IN_DIM), lambda i: (0, 0)),
        ],
        out_specs=[slab_spec] * 4,
        out_shape=[jax.ShapeDtypeStruct((8, N_EDGES), jnp.float32)] * 4,
    )(eaT, *xs, shT, w1, b1c, w2mT, b2rT)


def _final_body(p_ref, c_ref, na_ref, out_ref):
    s = p_ref[0] + p_ref[1]
    cnt = c_ref[0, :, 0:1] + c_ref[1, :, 0:1]
    out_ref[...] = s / jnp.maximum(cnt, EPS) + na_ref[...]


def _final(psum, pcnt, node_attr):
    return pl.pallas_call(
        _final_body,
        out_shape=jax.ShapeDtypeStruct((N_NODES, OUT_DIM), jnp.float32),
    )(psum, pcnt, node_attr)


def kernel(node_attr, edge_index, edge_attr, edge_sh, fc_w1, fc_b1, fc_w2,
           fc_b2):
    src = edge_index[0]
    dst = edge_index[1]
    xd = _sc_gather()(node_attr, dst)
    xs = tuple(xd[:, 8 * s:8 * s + 8].T for s in range(4))
    b1c = fc_b1.reshape(HID, 1)
    w2mT = ALPHA * fc_w2.reshape(IN_DIM, OUT_DIM, HID).transpose(2, 1, 0)\
        .reshape(HID * OUT_DIM, IN_DIM)
    b2rT = ALPHA * fc_b2.reshape(IN_DIM, OUT_DIM).T
    ss = _dense(edge_attr.T, xs, edge_sh.T, fc_w1, b1c, w2mT, b2rT)
    ts = tuple(s.T for s in ss)
    z8 = jnp.zeros((N_NODES, 8), jnp.float32)
    z16 = jnp.zeros((N_NODES, HID), jnp.float32)
    ones = jnp.ones((GC, HID), jnp.float32)
    psum, pcnt = _sc_scatter()(*ts, src, z8, z16, ones)
    return _final(psum, pcnt, node_attr)


# speedup vs baseline: 2.7058x; 2.7058x over previous
import functools

import jax
import jax.numpy as jnp
import numpy as np
from jax import lax
from jax.experimental import pallas as pl
from jax.experimental.pallas import tpu as pltpu
from jax.experimental.pallas import tpu_sc as plsc

N_NODES = 10000
N_EDGES = 160000
IN_DIM = 32
OUT_DIM = 32
NEF = 16
HID = 16
ALPHA = float(1.0 / np.sqrt(IN_DIM * 1))
EPS = float(jnp.finfo(jnp.float32).eps)

NC = 2
NS = 16
NW = NC * NS
EW = N_EDGES // NW
GC = 1000
NCHUNK = EW // GC
STRIPE = N_NODES // NS

@functools.lru_cache(maxsize=None)
def _get_mesh():
    return plsc.VectorSubcoreMesh(core_axis_name="c", subcore_axis_name="s",
                                  num_cores=NC, num_subcores=NS)


def _sc_gather_body(node_hbm, dst_hbm, out_hbm, idx_v, rows_v, sem):
    wid = lax.axis_index("s") * NC + lax.axis_index("c")
    for i in range(NCHUNK):
        base = wid * EW + i * GC
        pltpu.sync_copy(dst_hbm.at[pl.ds(base, GC)], idx_v)
        pltpu.async_copy(node_hbm.at[idx_v], rows_v, sem).wait()
        pltpu.sync_copy(rows_v, out_hbm.at[pl.ds(base, GC)])


@functools.lru_cache(maxsize=None)
def _sc_gather():
    return pl.kernel(
        _sc_gather_body,
        out_type=jax.ShapeDtypeStruct((N_EDGES, IN_DIM), jnp.float32),
        mesh=_get_mesh(),
        scratch_types=[
            pltpu.VMEM((GC,), jnp.int32),
            pltpu.VMEM((GC, IN_DIM), jnp.float32),
            pltpu.SemaphoreType.DMA,
        ],
        compiler_params=pltpu.CompilerParams(use_tc_tiling_on_sc=False,
                                             needs_layout_passes=False),
    )


def _sc_scatter_body(sum_hbm, src_hbm, z32_hbm, z16_hbm, ones_hbm,
                     psum_hbm, pcnt_hbm,
                     idx_v, val_v, ones_v, shared_sum, shared_cnt):
    cid = lax.axis_index("c")
    sid = lax.axis_index("s")
    row0 = sid * STRIPE
    pltpu.sync_copy(z32_hbm.at[pl.ds(row0, STRIPE)],
                    shared_sum.at[pl.ds(row0, STRIPE)])
    pltpu.sync_copy(z16_hbm.at[pl.ds(row0, STRIPE)],
                    shared_cnt.at[pl.ds(row0, STRIPE)])
    pltpu.sync_copy(ones_hbm, ones_v)
    plsc.subcore_barrier()
    wid = sid * NC + cid
    for i in range(NCHUNK):
        base = wid * EW + i * GC
        pltpu.sync_copy(src_hbm.at[pl.ds(base, GC)], idx_v)
        pltpu.sync_copy(sum_hbm.at[pl.ds(base, GC)], val_v)
        pltpu.sync_copy(val_v, shared_sum.at[idx_v], add=True)
        pltpu.sync_copy(ones_v, shared_cnt.at[idx_v], add=True)
    plsc.subcore_barrier()
    pltpu.sync_copy(shared_sum.at[pl.ds(row0, STRIPE)],
                    psum_hbm.at[cid, pl.ds(row0, STRIPE)])
    pltpu.sync_copy(shared_cnt.at[pl.ds(row0, STRIPE)],
                    pcnt_hbm.at[cid, pl.ds(row0, STRIPE)])


@functools.lru_cache(maxsize=None)
def _sc_scatter():
    return pl.kernel(
        _sc_scatter_body,
        out_type=(
            jax.ShapeDtypeStruct((NC, N_NODES, OUT_DIM), jnp.float32),
            jax.ShapeDtypeStruct((NC, N_NODES, HID), jnp.float32),
        ),
        mesh=_get_mesh(),
        scratch_types=[
            pltpu.VMEM((GC,), jnp.int32),
            pltpu.VMEM((GC, OUT_DIM), jnp.float32),
            pltpu.VMEM((GC, HID), jnp.float32),
            pltpu.VMEM_SHARED((N_NODES, OUT_DIM), jnp.float32),
            pltpu.VMEM_SHARED((N_NODES, HID), jnp.float32),
        ],
        compiler_params=pltpu.CompilerParams(use_tc_tiling_on_sc=False,
                                             needs_layout_passes=False),
    )


EB = 3200


def _dense_body(eaT_ref, xdT_ref, shT_ref, w1_ref, b1c_ref, w2mT_ref,
                b2rT_ref, out_ref):
    hT = jnp.maximum(
        jnp.dot(w1_ref[...], eaT_ref[...],
                preferred_element_type=jnp.float32) + b1c_ref[...], 0.0)
    xdsT = shT_ref[...] * xdT_ref[...]
    gT = jnp.dot(w2mT_ref[...], xdsT, preferred_element_type=jnp.float32)
    acc = jnp.dot(b2rT_ref[...], xdsT, preferred_element_type=jnp.float32)
    for j in range(HID):
        acc = acc + gT[j * OUT_DIM:(j + 1) * OUT_DIM, :] * hT[j:j + 1, :]
    out_ref[...] = acc


def _dense(eaT, xdT, shT, w1, b1c, w2mT, b2rT):
    return pl.pallas_call(
        _dense_body,
        grid=(N_EDGES // EB,),
        in_specs=[
            pl.BlockSpec((NEF, EB), lambda i: (0, i)),
            pl.BlockSpec((IN_DIM, EB), lambda i: (0, i)),
            pl.BlockSpec((1, EB), lambda i: (0, i)),
            pl.BlockSpec((NEF, NEF), lambda i: (0, 0)),
            pl.BlockSpec((HID, 1), lambda i: (0, 0)),
            pl.BlockSpec((HID * OUT_DIM, IN_DIM), lambda i: (0, 0)),
            pl.BlockSpec((OUT_DIM, IN_DIM), lambda i: (0, 0)),
        ],
        out_specs=pl.BlockSpec((OUT_DIM, EB), lambda i: (0, i)),
        out_shape=jax.ShapeDtypeStruct((OUT_DIM, N_EDGES), jnp.float32),
    )(eaT, xdT, shT, w1, b1c, w2mT, b2rT)


def _final_body(p_ref, c_ref, na_ref, out_ref):
    s = p_ref[0] + p_ref[1]
    cnt = c_ref[0, :, 0:1] + c_ref[1, :, 0:1]
    out_ref[...] = s / jnp.maximum(cnt, EPS) + na_ref[...]


def _final(psum, pcnt, node_attr):
    return pl.pallas_call(
        _final_body,
        out_shape=jax.ShapeDtypeStruct((N_NODES, OUT_DIM), jnp.float32),
    )(psum, pcnt, node_attr)


def kernel(node_attr, edge_index, edge_attr, edge_sh, fc_w1, fc_b1, fc_w2,
           fc_b2):
    src = edge_index[0]
    dst = edge_index[1]
    xd = _sc_gather()(node_attr, dst)
    b1c = fc_b1.reshape(HID, 1)
    w2mT = ALPHA * fc_w2.reshape(IN_DIM, OUT_DIM, HID).transpose(2, 1, 0)\
        .reshape(HID * OUT_DIM, IN_DIM)
    b2rT = ALPHA * fc_b2.reshape(IN_DIM, OUT_DIM).T
    summandT = _dense(edge_attr.T, xd.T, edge_sh.T, fc_w1, b1c, w2mT, b2rT)
    summand = summandT.T
    z32 = jnp.zeros((N_NODES, OUT_DIM), jnp.float32)
    z16 = jnp.zeros((N_NODES, HID), jnp.float32)
    ones = jnp.ones((GC, HID), jnp.float32)
    psum, pcnt = _sc_scatter()(summand, src, z32, z16, ones)
    return _final(psum, pcnt, node_attr)


# speedup vs baseline: 2.7431x; 1.0138x over previous
import functools

import jax
import jax.numpy as jnp
import numpy as np
from jax import lax
from jax.experimental import pallas as pl
from jax.experimental.pallas import tpu as pltpu
from jax.experimental.pallas import tpu_sc as plsc

N_NODES = 10000
N_EDGES = 160000
IN_DIM = 32
OUT_DIM = 32
NEF = 16
HID = 16
ALPHA = float(1.0 / np.sqrt(IN_DIM * 1))
EPS = float(jnp.finfo(jnp.float32).eps)

NC = 2
NS = 16
NW = NC * NS
EW = N_EDGES // NW
GC = 1000
NCHUNK = EW // GC
STRIPE = N_NODES // NS

@functools.lru_cache(maxsize=None)
def _get_mesh():
    return plsc.VectorSubcoreMesh(core_axis_name="c", subcore_axis_name="s",
                                  num_cores=NC, num_subcores=NS)


def _sc_gather_body(node_hbm, ei_hbm, out_hbm, idx_v, rows_v, sem):
    wid = lax.axis_index("s") * NC + lax.axis_index("c")
    for i in range(NCHUNK):
        base = wid * EW + i * GC
        pltpu.sync_copy(ei_hbm.at[1, pl.ds(base, GC)], idx_v)
        pltpu.async_copy(node_hbm.at[idx_v], rows_v, sem).wait()
        pltpu.sync_copy(rows_v, out_hbm.at[pl.ds(base, GC)])


@functools.lru_cache(maxsize=None)
def _sc_gather():
    return pl.kernel(
        _sc_gather_body,
        out_type=jax.ShapeDtypeStruct((N_EDGES, IN_DIM), jnp.float32),
        mesh=_get_mesh(),
        scratch_types=[
            pltpu.VMEM((GC,), jnp.int32),
            pltpu.VMEM((GC, IN_DIM), jnp.float32),
            pltpu.SemaphoreType.DMA,
        ],
        compiler_params=pltpu.CompilerParams(use_tc_tiling_on_sc=False,
                                             needs_layout_passes=False),
    )


def _sc_scatter_body(sum_hbm, ei_hbm, z32_hbm, z16_hbm, ones_hbm,
                     psum_hbm, pcnt_hbm,
                     idx_v, val_v, ones_v, shared_sum, shared_cnt):
    cid = lax.axis_index("c")
    sid = lax.axis_index("s")
    row0 = sid * STRIPE
    pltpu.sync_copy(z32_hbm.at[pl.ds(row0, STRIPE)],
                    shared_sum.at[pl.ds(row0, STRIPE)])
    pltpu.sync_copy(z16_hbm.at[pl.ds(row0, STRIPE)],
                    shared_cnt.at[pl.ds(row0, STRIPE)])
    pltpu.sync_copy(ones_hbm, ones_v)
    plsc.subcore_barrier()
    wid = sid * NC + cid
    for i in range(NCHUNK):
        base = wid * EW + i * GC
        pltpu.sync_copy(ei_hbm.at[0, pl.ds(base, GC)], idx_v)
        pltpu.sync_copy(sum_hbm.at[pl.ds(base, GC)], val_v)
        pltpu.sync_copy(val_v, shared_sum.at[idx_v], add=True)
        pltpu.sync_copy(ones_v, shared_cnt.at[idx_v], add=True)
    plsc.subcore_barrier()
    pltpu.sync_copy(shared_sum.at[pl.ds(row0, STRIPE)],
                    psum_hbm.at[cid, pl.ds(row0, STRIPE)])
    pltpu.sync_copy(shared_cnt.at[pl.ds(row0, STRIPE)],
                    pcnt_hbm.at[cid, pl.ds(row0, STRIPE)])


@functools.lru_cache(maxsize=None)
def _sc_scatter():
    return pl.kernel(
        _sc_scatter_body,
        out_type=(
            jax.ShapeDtypeStruct((NC, N_NODES, OUT_DIM), jnp.float32),
            jax.ShapeDtypeStruct((NC, N_NODES, HID), jnp.float32),
        ),
        mesh=_get_mesh(),
        scratch_types=[
            pltpu.VMEM((GC,), jnp.int32),
            pltpu.VMEM((GC, OUT_DIM), jnp.float32),
            pltpu.VMEM((GC, HID), jnp.float32),
            pltpu.VMEM_SHARED((N_NODES, OUT_DIM), jnp.float32),
            pltpu.VMEM_SHARED((N_NODES, HID), jnp.float32),
        ],
        compiler_params=pltpu.CompilerParams(use_tc_tiling_on_sc=False,
                                             needs_layout_passes=False),
    )


EB = 3200


def _dense_body(eaT_ref, xdT_ref, shT_ref, w1_ref, b1c_ref, w2mT_ref,
                b2rT_ref, out_ref):
    hT = jnp.maximum(
        jnp.dot(w1_ref[...], eaT_ref[...],
                preferred_element_type=jnp.float32) + b1c_ref[...], 0.0)
    xdsT = shT_ref[...] * xdT_ref[...]
    gT = jnp.dot(w2mT_ref[...], xdsT.astype(jnp.bfloat16),
                 preferred_element_type=jnp.float32)
    acc = jnp.dot(b2rT_ref[...], xdsT, preferred_element_type=jnp.float32)
    for j in range(HID):
        acc = acc + gT[j * OUT_DIM:(j + 1) * OUT_DIM, :] * hT[j:j + 1, :]
    out_ref[...] = acc


def _dense(eaT, xdT, shT, w1, b1c, w2mT, b2rT):
    return pl.pallas_call(
        _dense_body,
        grid=(N_EDGES // EB,),
        in_specs=[
            pl.BlockSpec((NEF, EB), lambda i: (0, i)),
            pl.BlockSpec((IN_DIM, EB), lambda i: (0, i)),
            pl.BlockSpec((1, EB), lambda i: (0, i)),
            pl.BlockSpec((NEF, NEF), lambda i: (0, 0)),
            pl.BlockSpec((HID, 1), lambda i: (0, 0)),
            pl.BlockSpec((HID * OUT_DIM, IN_DIM), lambda i: (0, 0)),
            pl.BlockSpec((OUT_DIM, IN_DIM), lambda i: (0, 0)),
        ],
        out_specs=pl.BlockSpec((OUT_DIM, EB), lambda i: (0, i)),
        out_shape=jax.ShapeDtypeStruct((OUT_DIM, N_EDGES), jnp.float32),
    )(eaT, xdT, shT, w1, b1c, w2mT, b2rT)


def _final_body(p_ref, c_ref, na_ref, out_ref):
    s = p_ref[0] + p_ref[1]
    cnt = c_ref[0, :, 0:1] + c_ref[1, :, 0:1]
    out_ref[...] = s / jnp.maximum(cnt, EPS) + na_ref[...]


def _final(psum, pcnt, node_attr):
    return pl.pallas_call(
        _final_body,
        out_shape=jax.ShapeDtypeStruct((N_NODES, OUT_DIM), jnp.float32),
    )(psum, pcnt, node_attr)


def kernel(node_attr, edge_index, edge_attr, edge_sh, fc_w1, fc_b1, fc_w2,
           fc_b2):
    xd = _sc_gather()(node_attr, edge_index)
    b1c = fc_b1.reshape(HID, 1)
    w2mT = (ALPHA * fc_w2.reshape(IN_DIM, OUT_DIM, HID).transpose(2, 1, 0)
            .reshape(HID * OUT_DIM, IN_DIM)).astype(jnp.bfloat16)
    b2rT = ALPHA * fc_b2.reshape(IN_DIM, OUT_DIM).T
    summandT = _dense(edge_attr.T, xd.T, edge_sh.T, fc_w1, b1c, w2mT, b2rT)
    summand = summandT.T
    z32 = jnp.zeros((N_NODES, OUT_DIM), jnp.float32)
    z16 = jnp.zeros((N_NODES, HID), jnp.float32)
    ones = jnp.ones((GC, HID), jnp.float32)
    psum, pcnt = _sc_scatter()(summand, edge_index, z32, z16, ones)
    return _final(psum, pcnt, node_attr)


# speedup vs baseline: 2.8733x; 1.0475x over previous
import functools

import jax
import jax.numpy as jnp
import numpy as np
from jax import lax
from jax.experimental import pallas as pl
from jax.experimental.pallas import tpu as pltpu
from jax.experimental.pallas import tpu_sc as plsc

N_NODES = 10000
N_EDGES = 160000
IN_DIM = 32
OUT_DIM = 32
NEF = 16
HID = 16
ALPHA = float(1.0 / np.sqrt(IN_DIM * 1))
EPS = float(jnp.finfo(jnp.float32).eps)

NC = 2
NS = 16
NW = NC * NS
EW = N_EDGES // NW
GC = 1000
NCHUNK = EW // GC
STRIPE = N_NODES // NS

@functools.lru_cache(maxsize=None)
def _get_mesh():
    return plsc.VectorSubcoreMesh(core_axis_name="c", subcore_axis_name="s",
                                  num_cores=NC, num_subcores=NS)


@functools.lru_cache(maxsize=None)
def _sc_gather(eoff, ne):
    ew = ne // NW
    nch = ew // GC

    def body(node_hbm, ei_hbm, out_hbm, idx_v, rows_v, sem):
        wid = lax.axis_index("s") * NC + lax.axis_index("c")
        for i in range(nch):
            base = wid * ew + i * GC
            pltpu.sync_copy(ei_hbm.at[1, pl.ds(eoff + base, GC)], idx_v)
            pltpu.async_copy(node_hbm.at[idx_v], rows_v, sem).wait()
            pltpu.sync_copy(rows_v, out_hbm.at[pl.ds(base, GC)])

    return pl.kernel(
        body,
        out_type=jax.ShapeDtypeStruct((ne, IN_DIM), jnp.float32),
        mesh=_get_mesh(),
        scratch_types=[
            pltpu.VMEM((GC,), jnp.int32),
            pltpu.VMEM((GC, IN_DIM), jnp.float32),
            pltpu.SemaphoreType.DMA,
        ],
        compiler_params=pltpu.CompilerParams(use_tc_tiling_on_sc=False,
                                             needs_layout_passes=False),
    )


@functools.lru_cache(maxsize=None)
def _sc_scatter(eoff, ne):
    ew = ne // NW
    nch = ew // GC

    def body(sum_hbm, ei_hbm, z32_hbm, z16_hbm, ones_hbm,
             psum_hbm, pcnt_hbm,
             idx_v, val_v, ones_v, shared_sum, shared_cnt):
        cid = lax.axis_index("c")
        sid = lax.axis_index("s")
        row0 = sid * STRIPE
        pltpu.sync_copy(z32_hbm.at[pl.ds(row0, STRIPE)],
                        shared_sum.at[pl.ds(row0, STRIPE)])
        pltpu.sync_copy(z16_hbm.at[pl.ds(row0, STRIPE)],
                        shared_cnt.at[pl.ds(row0, STRIPE)])
        pltpu.sync_copy(ones_hbm, ones_v)
        plsc.subcore_barrier()
        wid = sid * NC + cid
        for i in range(nch):
            base = wid * ew + i * GC
            pltpu.sync_copy(ei_hbm.at[0, pl.ds(eoff + base, GC)], idx_v)
            pltpu.sync_copy(sum_hbm.at[pl.ds(base, GC)], val_v)
            pltpu.sync_copy(val_v, shared_sum.at[idx_v], add=True)
            pltpu.sync_copy(ones_v, shared_cnt.at[idx_v], add=True)
        plsc.subcore_barrier()
        pltpu.sync_copy(shared_sum.at[pl.ds(row0, STRIPE)],
                        psum_hbm.at[cid, pl.ds(row0, STRIPE)])
        pltpu.sync_copy(shared_cnt.at[pl.ds(row0, STRIPE)],
                        pcnt_hbm.at[cid, pl.ds(row0, STRIPE)])

    return pl.kernel(
        body,
        out_type=(
            jax.ShapeDtypeStruct((NC, N_NODES, OUT_DIM), jnp.float32),
            jax.ShapeDtypeStruct((NC, N_NODES, HID), jnp.float32),
        ),
        mesh=_get_mesh(),
        scratch_types=[
            pltpu.VMEM((GC,), jnp.int32),
            pltpu.VMEM((GC, OUT_DIM), jnp.float32),
            pltpu.VMEM((GC, HID), jnp.float32),
            pltpu.VMEM_SHARED((N_NODES, OUT_DIM), jnp.float32),
            pltpu.VMEM_SHARED((N_NODES, HID), jnp.float32),
        ],
        compiler_params=pltpu.CompilerParams(use_tc_tiling_on_sc=False,
                                             needs_layout_passes=False),
    )


EB = 3200


def _dense_body(eaT_ref, xdT_ref, shT_ref, w1_ref, b1c_ref, w2mT_ref,
                b2rT_ref, out_ref):
    hT = jnp.maximum(
        jnp.dot(w1_ref[...], eaT_ref[...],
                preferred_element_type=jnp.float32) + b1c_ref[...], 0.0)
    xdsT = shT_ref[...] * xdT_ref[...]
    gT = jnp.dot(w2mT_ref[...], xdsT.astype(jnp.bfloat16),
                 preferred_element_type=jnp.float32)
    acc = jnp.dot(b2rT_ref[...], xdsT, preferred_element_type=jnp.float32)
    for j in range(HID):
        acc = acc + gT[j * OUT_DIM:(j + 1) * OUT_DIM, :] * hT[j:j + 1, :]
    out_ref[...] = acc


def _dense(eaT, xdT, shT, w1, b1c, w2mT, b2rT, eoff, ne):
    boff = eoff // EB
    return pl.pallas_call(
        _dense_body,
        grid=(ne // EB,),
        in_specs=[
            pl.BlockSpec((NEF, EB), lambda i: (0, i + boff)),
            pl.BlockSpec((IN_DIM, EB), lambda i: (0, i)),
            pl.BlockSpec((1, EB), lambda i: (0, i + boff)),
            pl.BlockSpec((NEF, NEF), lambda i: (0, 0)),
            pl.BlockSpec((HID, 1), lambda i: (0, 0)),
            pl.BlockSpec((HID * OUT_DIM, IN_DIM), lambda i: (0, 0)),
            pl.BlockSpec((OUT_DIM, IN_DIM), lambda i: (0, 0)),
        ],
        out_specs=pl.BlockSpec((OUT_DIM, EB), lambda i: (0, i)),
        out_shape=jax.ShapeDtypeStruct((OUT_DIM, ne), jnp.float32),
    )(eaT, xdT, shT, w1, b1c, w2mT, b2rT)


def _final_body(pa_ref, pb_ref, ca_ref, cb_ref, na_ref, out_ref):
    s = pa_ref[0] + pa_ref[1] + pb_ref[0] + pb_ref[1]
    cnt = (ca_ref[0, :, 0:1] + ca_ref[1, :, 0:1]
           + cb_ref[0, :, 0:1] + cb_ref[1, :, 0:1])
    out_ref[...] = s / jnp.maximum(cnt, EPS) + na_ref[...]


def _final(pa, pb, ca, cb, node_attr):
    return pl.pallas_call(
        _final_body,
        out_shape=jax.ShapeDtypeStruct((N_NODES, OUT_DIM), jnp.float32),
    )(pa, pb, ca, cb, node_attr)


def kernel(node_attr, edge_index, edge_attr, edge_sh, fc_w1, fc_b1, fc_w2,
           fc_b2):
    b1c = fc_b1.reshape(HID, 1)
    w2mT = (ALPHA * fc_w2.reshape(IN_DIM, OUT_DIM, HID).transpose(2, 1, 0)
            .reshape(HID * OUT_DIM, IN_DIM)).astype(jnp.bfloat16)
    b2rT = ALPHA * fc_b2.reshape(IN_DIM, OUT_DIM).T
    z32 = jnp.zeros((N_NODES, OUT_DIM), jnp.float32)
    z16 = jnp.zeros((N_NODES, HID), jnp.float32)
    ones = jnp.ones((GC, HID), jnp.float32)
    eaT = edge_attr.T
    shT = edge_sh.T
    halves = ((0, 96000), (96000, 64000))
    parts = []
    for eoff, ne in halves:
        xd = _sc_gather(eoff, ne)(node_attr, edge_index)
        sT = _dense(eaT, xd.T, shT, fc_w1, b1c, w2mT, b2rT, eoff, ne)
        parts.append(
            _sc_scatter(eoff, ne)(sT.T, edge_index, z32, z16, ones))
    (pa, ca), (pb, cb) = parts
    return _final(pa, pb, ca, cb, node_attr)


# speedup vs baseline: 2.9273x; 1.0188x over previous
import functools

import jax
import jax.numpy as jnp
import numpy as np
from jax import lax
from jax.experimental import pallas as pl
from jax.experimental.pallas import tpu as pltpu
from jax.experimental.pallas import tpu_sc as plsc

N_NODES = 10000
N_EDGES = 160000
IN_DIM = 32
OUT_DIM = 32
NEF = 16
HID = 16
ALPHA = float(1.0 / np.sqrt(IN_DIM * 1))
EPS = float(jnp.finfo(jnp.float32).eps)

NC = 2
NS = 16
NW = NC * NS
EW = N_EDGES // NW
GC = 1000
NCHUNK = EW // GC
STRIPE = N_NODES // NS

@functools.lru_cache(maxsize=None)
def _get_mesh():
    return plsc.VectorSubcoreMesh(core_axis_name="c", subcore_axis_name="s",
                                  num_cores=NC, num_subcores=NS)


@functools.lru_cache(maxsize=None)
def _sc_gather(eoff, ne):
    ew = ne // NW
    nch = ew // GC

    def body(node_hbm, ei_hbm, out_hbm, idx_v, rows_v, sem):
        wid = lax.axis_index("s") * NC + lax.axis_index("c")
        for i in range(nch):
            base = wid * ew + i * GC
            pltpu.sync_copy(ei_hbm.at[1, pl.ds(eoff + base, GC)], idx_v)
            pltpu.async_copy(node_hbm.at[idx_v], rows_v, sem).wait()
            pltpu.sync_copy(rows_v, out_hbm.at[pl.ds(base, GC)])

    return pl.kernel(
        body,
        out_type=jax.ShapeDtypeStruct((ne, IN_DIM), jnp.float32),
        mesh=_get_mesh(),
        scratch_types=[
            pltpu.VMEM((GC,), jnp.int32),
            pltpu.VMEM((GC, IN_DIM), jnp.float32),
            pltpu.SemaphoreType.DMA,
        ],
        compiler_params=pltpu.CompilerParams(use_tc_tiling_on_sc=False,
                                             needs_layout_passes=False),
    )


@functools.lru_cache(maxsize=None)
def _sc_scatter(eoff, ne):
    ew = ne // NW
    nch = ew // GC

    def body(sum_hbm, ei_hbm, z32_hbm, z16_hbm, ones_hbm,
             psum_hbm, pcnt_hbm,
             idx_v, val_v, ones_v, shared_sum, shared_cnt):
        cid = lax.axis_index("c")
        sid = lax.axis_index("s")
        row0 = sid * STRIPE
        pltpu.sync_copy(z32_hbm.at[pl.ds(row0, STRIPE)],
                        shared_sum.at[pl.ds(row0, STRIPE)])
        pltpu.sync_copy(z16_hbm.at[pl.ds(row0, STRIPE)],
                        shared_cnt.at[pl.ds(row0, STRIPE)])
        pltpu.sync_copy(ones_hbm, ones_v)
        plsc.subcore_barrier()
        wid = sid * NC + cid
        for i in range(nch):
            base = wid * ew + i * GC
            pltpu.sync_copy(ei_hbm.at[0, pl.ds(eoff + base, GC)], idx_v)
            pltpu.sync_copy(sum_hbm.at[pl.ds(base, GC)], val_v)
            pltpu.sync_copy(val_v, shared_sum.at[idx_v], add=True)
            pltpu.sync_copy(ones_v, shared_cnt.at[idx_v], add=True)
        plsc.subcore_barrier()
        pltpu.sync_copy(shared_sum.at[pl.ds(row0, STRIPE)],
                        psum_hbm.at[cid, pl.ds(row0, STRIPE)])
        pltpu.sync_copy(shared_cnt.at[pl.ds(row0, STRIPE)],
                        pcnt_hbm.at[cid, pl.ds(row0, STRIPE)])

    return pl.kernel(
        body,
        out_type=(
            jax.ShapeDtypeStruct((NC, N_NODES, OUT_DIM), jnp.float32),
            jax.ShapeDtypeStruct((NC, N_NODES, HID), jnp.float32),
        ),
        mesh=_get_mesh(),
        scratch_types=[
            pltpu.VMEM((GC,), jnp.int32),
            pltpu.VMEM((GC, OUT_DIM), jnp.float32),
            pltpu.VMEM((GC, HID), jnp.float32),
            pltpu.VMEM_SHARED((N_NODES, OUT_DIM), jnp.float32),
            pltpu.VMEM_SHARED((N_NODES, HID), jnp.float32),
        ],
        compiler_params=pltpu.CompilerParams(use_tc_tiling_on_sc=False,
                                             needs_layout_passes=False),
    )


EB = 6400


def _dense_body(eaT_ref, xdT_ref, shT_ref, w1_ref, b1c_ref, w2mT_ref,
                b2rT_ref, out_ref):
    hT = jnp.maximum(
        jnp.dot(w1_ref[...], eaT_ref[...],
                preferred_element_type=jnp.float32) + b1c_ref[...], 0.0)
    xdsT = shT_ref[...] * xdT_ref[...]
    gT = jnp.dot(w2mT_ref[...], xdsT.astype(jnp.bfloat16),
                 preferred_element_type=jnp.float32)
    acc = jnp.dot(b2rT_ref[...], xdsT, preferred_element_type=jnp.float32)
    for j in range(HID):
        acc = acc + gT[j * OUT_DIM:(j + 1) * OUT_DIM, :] * hT[j:j + 1, :]
    out_ref[...] = acc


def _dense(eaT, xdT, shT, w1, b1c, w2mT, b2rT, eoff, ne):
    boff = eoff // EB
    return pl.pallas_call(
        _dense_body,
        grid=(ne // EB,),
        in_specs=[
            pl.BlockSpec((NEF, EB), lambda i: (0, i + boff)),
            pl.BlockSpec((IN_DIM, EB), lambda i: (0, i)),
            pl.BlockSpec((1, EB), lambda i: (0, i + boff)),
            pl.BlockSpec((NEF, NEF), lambda i: (0, 0)),
            pl.BlockSpec((HID, 1), lambda i: (0, 0)),
            pl.BlockSpec((HID * OUT_DIM, IN_DIM), lambda i: (0, 0)),
            pl.BlockSpec((OUT_DIM, IN_DIM), lambda i: (0, 0)),
        ],
        out_specs=pl.BlockSpec((OUT_DIM, EB), lambda i: (0, i)),
        out_shape=jax.ShapeDtypeStruct((OUT_DIM, ne), jnp.float32),
    )(eaT, xdT, shT, w1, b1c, w2mT, b2rT)


def _final_body(pa_ref, pb_ref, ca_ref, cb_ref, na_ref, out_ref):
    s = pa_ref[0] + pa_ref[1] + pb_ref[0] + pb_ref[1]
    cnt = (ca_ref[0, :, 0:1] + ca_ref[1, :, 0:1]
           + cb_ref[0, :, 0:1] + cb_ref[1, :, 0:1])
    out_ref[...] = s / jnp.maximum(cnt, EPS) + na_ref[...]


def _final(pa, pb, ca, cb, node_attr):
    return pl.pallas_call(
        _final_body,
        out_shape=jax.ShapeDtypeStruct((N_NODES, OUT_DIM), jnp.float32),
    )(pa, pb, ca, cb, node_attr)


def kernel(node_attr, edge_index, edge_attr, edge_sh, fc_w1, fc_b1, fc_w2,
           fc_b2):
    b1c = fc_b1.reshape(HID, 1)
    w2mT = (ALPHA * fc_w2.reshape(IN_DIM, OUT_DIM, HID).transpose(2, 1, 0)
            .reshape(HID * OUT_DIM, IN_DIM)).astype(jnp.bfloat16)
    b2rT = ALPHA * fc_b2.reshape(IN_DIM, OUT_DIM).T
    z32 = jnp.zeros((N_NODES, OUT_DIM), jnp.float32)
    z16 = jnp.zeros((N_NODES, HID), jnp.float32)
    ones = jnp.ones((GC, HID), jnp.float32)
    eaT = edge_attr.T
    shT = edge_sh.T
    halves = ((0, 96000), (96000, 64000))
    parts = []
    for eoff, ne in halves:
        xd = _sc_gather(eoff, ne)(node_attr, edge_index)
        sT = _dense(eaT, xd.T, shT, fc_w1, b1c, w2mT, b2rT, eoff, ne)
        parts.append(
            _sc_scatter(eoff, ne)(sT.T, edge_index, z32, z16, ones))
    (pa, ca), (pb, cb) = parts
    return _final(pa, pb, ca, cb, node_attr)
